# baseline (speedup 1.0000x reference)
import jax
import jax.numpy as jnp
from jax.experimental import pallas as pl
from jax.experimental.pallas import tpu as pltpu
from jax.experimental.pallas import tpu_sc as plsc

VOCAB = 100000
EMB = 128
HID = 128
B = 1024
T = 200
G = 3 * HID


_GATHER_WINDOW = 128


def _sc_gather(table, idx):
    n = idx.shape[1]
    mesh = plsc.VectorSubcoreMesh(core_axis_name="c", subcore_axis_name="s")

    def run(table, idx):
        @pl.kernel(
            out_type=jax.ShapeDtypeStruct((n, EMB), table.dtype),
            mesh=mesh,
        )
        def k(tbl_hbm, idx_hbm, out_hbm):
            def body(i_vmem, o_vmem):
                pltpu.sync_copy(tbl_hbm.at[i_vmem.at[0]], o_vmem)

            pltpu.emit_pipeline(
                body,
                grid=(n // _GATHER_WINDOW,),
                in_specs=[
                    pl.BlockSpec((1, _GATHER_WINDOW), lambda i: (0, i)),
                ],
                out_specs=[
                    pl.BlockSpec((_GATHER_WINDOW, EMB), lambda i: (i, 0)),
                ],
                core_axis_name=("c", "s"),
                dimension_semantics=(pltpu.PARALLEL,),
            )(idx_hbm, out_hbm)

        return k(table, idx)

    return run(table, idx)


_TS = 8
_NB = 2
_BB = B // _NB


def _gru_body(emb_ref, wih_ref, whh_ref, bih_ref, bhh_ref, out_ref, h_ref):
    t = pl.program_id(1)

    @pl.when(t == 0)
    def _init():
        h_ref[...] = jnp.zeros_like(h_ref)

    h = h_ref[...]
    wih = wih_ref[...]
    whh = whh_ref[...]
    bih = bih_ref[...]
    bhh = bhh_ref[...]
    for s in range(_TS):
        e = emb_ref[s].astype(jnp.bfloat16)
        gi = jnp.dot(e, wih, preferred_element_type=jnp.float32) + bih
        gh = jnp.dot(h.astype(jnp.bfloat16), whh,
                     preferred_element_type=jnp.float32) + bhh
        r = jax.nn.sigmoid(gi[:, :HID] + gh[:, :HID])
        z = jax.nn.sigmoid(gi[:, HID:2 * HID] + gh[:, HID:2 * HID])
        nn = jnp.tanh(gi[:, 2 * HID:] + r * gh[:, 2 * HID:])
        h = (1.0 - z) * nn + z * h
        out_ref[:, s, :] = h
    h_ref[...] = h


def _tc_gru(emb, w_ih, w_hh, b_ih, b_hh):
    return pl.pallas_call(
        _gru_body,
        grid=(_NB, T // _TS),
        in_specs=[
            pl.BlockSpec((_TS, _BB, EMB), lambda j, t: (t, j, 0)),
            pl.BlockSpec((EMB, G), lambda j, t: (0, 0)),
            pl.BlockSpec((HID, G), lambda j, t: (0, 0)),
            pl.BlockSpec((1, G), lambda j, t: (0, 0)),
            pl.BlockSpec((1, G), lambda j, t: (0, 0)),
        ],
        out_specs=pl.BlockSpec((_BB, _TS, HID), lambda j, t: (j, t, 0)),
        out_shape=jax.ShapeDtypeStruct((B, T, HID), jnp.float32),
        scratch_shapes=[pltpu.VMEM((_BB, HID), jnp.float32)],
        compiler_params=pltpu.CompilerParams(
            dimension_semantics=("parallel", "arbitrary"),
        ),
    )(emb, w_ih.astype(jnp.bfloat16), w_hh.astype(jnp.bfloat16),
      b_ih.reshape(1, G), b_hh.reshape(1, G))


def kernel(x, table, W_ih, W_hh, b_ih, b_hh):
    idx = x.astype(jnp.int32).T.reshape(1, T * B)
    emb = _sc_gather(table, idx).reshape(T, B, EMB)
    return _tc_gru(emb, W_ih, W_hh, b_ih, b_hh)


# speedup vs baseline: 1.3850x; 1.3850x over previous
import functools

import jax
import jax.numpy as jnp
from jax.experimental import pallas as pl
from jax.experimental.pallas import tpu as pltpu
from jax.experimental.pallas import tpu_sc as plsc

VOCAB = 100000
EMB = 128
HID = 128
B = 1024
T = 200
G = 3 * HID

_GATHER_WINDOW = 128


def _sc_gather(table, idx):
    n = idx.shape[1]
    mesh = plsc.VectorSubcoreMesh(core_axis_name="c", subcore_axis_name="s")

    @pl.kernel(
        out_type=jax.ShapeDtypeStruct((n, EMB), table.dtype),
        mesh=mesh,
    )
    def k(tbl_hbm, idx_hbm, out_hbm):
        def body(i_vmem, o_vmem):
            pltpu.sync_copy(tbl_hbm.at[i_vmem.at[0]], o_vmem)

        pltpu.emit_pipeline(
            body,
            grid=(n // _GATHER_WINDOW,),
            in_specs=[
                pl.BlockSpec((1, _GATHER_WINDOW), lambda i: (0, i)),
            ],
            out_specs=[
                pl.BlockSpec((_GATHER_WINDOW, EMB), lambda i: (i, 0)),
            ],
            core_axis_name=("c", "s"),
            dimension_semantics=(pltpu.PARALLEL,),
        )(idx_hbm, out_hbm)

    return k(table, idx)


_C = 5
_TCH = T // _C
_TS = 8
_NSTEP = _TCH // _TS
_NB = 2
_BB = B // _NB


def _gru_chunk_body(first, *refs):
    if first:
        emb_ref, wcat_ref, bias_ref, out_ref, hout_ref, h_ref = refs
        hin_ref = None
    else:
        (emb_ref, wcat_ref, bias_ref, hin_ref, _outprev,
         out_ref, hout_ref, h_ref) = refs
    t = pl.program_id(1)

    @pl.when(t == 0)
    def _init():
        if first:
            h_ref[...] = jnp.zeros_like(h_ref)
        else:
            h_ref[...] = hin_ref[...]

    h = h_ref[...]
    wcat = wcat_ref[...]
    bias = bias_ref[...]
    for s in range(_TS):
        eh = jnp.concatenate([emb_ref[s], h], axis=1)
        g = jnp.dot(eh, wcat, preferred_element_type=jnp.float32) + bias
        ur = jnp.tanh(g[:, :HID])
        uz = jnp.tanh(g[:, HID:2 * HID])
        ch = g[:, 3 * HID:]
        nn = jnp.tanh(g[:, 2 * HID:3 * HID] + ch + ur * ch)
        d = h - nn
        h = nn + 0.5 * (d + uz * d)
        out_ref[:, s, :] = h
    h_ref[...] = h
    hout_ref[...] = h


_OUT_SHAPES = [
    jax.ShapeDtypeStruct((B, T, HID), jnp.float32),
    jax.ShapeDtypeStruct((B, HID), jnp.float32),
]
_CPARAMS = pltpu.CompilerParams(dimension_semantics=("arbitrary", "arbitrary"))


def _scan_chunk(c, emb_c, wcat, bias, h_in, out_sofar):
    first = c == 0
    base_specs = [
        pl.BlockSpec((_TS, _BB, EMB), lambda j, t: (t, j, 0)),
        pl.BlockSpec((EMB + HID, 4 * HID), lambda j, t: (0, 0)),
        pl.BlockSpec((1, 4 * HID), lambda j, t: (0, 0)),
    ]
    out_specs = [
        pl.BlockSpec((_BB, _TS, HID), lambda j, t: (j, t + c * _NSTEP, 0)),
        pl.BlockSpec((_BB, HID), lambda j, t: (j, 0)),
    ]
    if first:
        return pl.pallas_call(
            functools.partial(_gru_chunk_body, True),
            grid=(_NB, _NSTEP),
            in_specs=base_specs,
            out_specs=out_specs,
            out_shape=_OUT_SHAPES,
            scratch_shapes=[pltpu.VMEM((_BB, HID), jnp.float32)],
            compiler_params=_CPARAMS,
        )(emb_c, wcat, bias)
    return pl.pallas_call(
        functools.partial(_gru_chunk_body, False),
        grid=(_NB, _NSTEP),
        in_specs=base_specs + [
            pl.BlockSpec((_BB, HID), lambda j, t: (j, 0)),
            pl.BlockSpec(memory_space=pl.ANY),
        ],
        out_specs=out_specs,
        out_shape=_OUT_SHAPES,
        scratch_shapes=[pltpu.VMEM((_BB, HID), jnp.float32)],
        input_output_aliases={4: 0},
        compiler_params=_CPARAMS,
    )(emb_c, wcat, bias, h_in, out_sofar)


def kernel(x, table, W_ih, W_hh, b_ih, b_hh):
    idx = x.astype(jnp.int32).T.reshape(1, T * B)
    z_eh = jnp.zeros((EMB, HID), jnp.float32)
    z_hh = jnp.zeros((HID, HID), jnp.float32)
    top = jnp.concatenate(
        [0.5 * W_ih[:, :2 * HID], W_ih[:, 2 * HID:], z_eh], axis=1)
    bot = jnp.concatenate(
        [0.5 * W_hh[:, :2 * HID], z_hh, 0.5 * W_hh[:, 2 * HID:]], axis=1)
    wcat = jnp.concatenate([top, bot], axis=0)
    bias = jnp.concatenate(
        [0.5 * (b_ih + b_hh)[:2 * HID], b_ih[2 * HID:],
         0.5 * b_hh[2 * HID:]]).reshape(1, 4 * HID)
    embs = [
        _sc_gather(table, idx[:, c * _TCH * B:(c + 1) * _TCH * B])
        .reshape(_TCH, B, EMB)
        for c in range(_C)
    ]
    out, h = _scan_chunk(0, embs[0], wcat, bias, None, None)
    for c in range(1, _C):
        out, h = _scan_chunk(c, embs[c], wcat, bias, h, out)
    return out
